# Initial kernel scaffold; baseline (speedup 1.0000x reference)
import math

import numpy as np
import jax
import jax.numpy as jnp
from jax import lax
from jax.experimental import pallas as pl
from jax.experimental.pallas import tpu as pltpu
from jax.experimental.pallas import tpu_sc as plsc

N = 10000
E = 320000
D = 128
NCORE = 2
NSUB = 16
NW = NCORE * NSUB
EPW = E // NW
ROWS = 125
RLEN = 80
NPS = N // NSUB

_LOG_FACT = [float(math.lgamma(k + 1)) for k in range(10)]


def _prep_body(h_ref, w_ref, a_ref, d_ref, z_ref, s_ref, p_ref):
    z = jnp.dot(h_ref[...], w_ref[...].T, preferred_element_type=jnp.float32)
    z_ref[...] = z
    s_ref[...] = jnp.dot(z, a_ref[...], preferred_element_type=jnp.float32)
    dist = d_ref[...]
    mu = jnp.sum(dist).astype(jnp.float32) / np.float32(E)
    kv = lax.shift_right_logical(dist, 1)
    lf = jnp.zeros(dist.shape, jnp.float32)
    for j in range(1, 10):
        lf = jnp.where(kv == j, np.float32(_LOG_FACT[j]), lf)
    kf = kv.astype(jnp.float32)
    p_ref[...] = jnp.exp(kf * jnp.log(mu) - mu - lf)


def _tc_prep(h, fc_W, attn_T, dist2d):
    return pl.pallas_call(
        _prep_body,
        out_shape=(
            jax.ShapeDtypeStruct((N, D), jnp.float32),
            jax.ShapeDtypeStruct((N, 2), jnp.float32),
            jax.ShapeDtypeStruct((E // D, D), jnp.float32),
        ),
    )(h, fc_W, attn_T, dist2d)


def _lane_bcast(v, lane):
    idx = jnp.full((16, 1), lane, dtype=jnp.int32)
    dn = lax.GatherDimensionNumbers(
        offset_dims=(), collapsed_slice_dims=(0,), start_index_map=(0,))
    return lax.gather(v, idx, dn, (1,),
                      mode=lax.GatherScatterMode.PROMISE_IN_BOUNDS)


def _sc_body(z_hbm, s_hbm, src_hbm, dst_hbm, pr_hbm,
             out_hbm, den_hbm,
             s_l, src_l, dst_l, pr_l, prow, zbuf, zero2d, zero1d,
             out_sh, den_sh):
    cid = lax.axis_index("c")
    sid = lax.axis_index("s")
    wid = sid * NCORE + cid

    zv16 = jnp.zeros((16,), jnp.float32)

    @pl.loop(0, 125)
    def _(j):
        for r in range(D // 16):
            zero2d[j, pl.ds(r * 16, 16)] = zv16

    @pl.loop(0, 2000, step=16)
    def _(i):
        zero1d[pl.ds(i, 16), 0] = zv16

    for b in range(5):
        pltpu.sync_copy(zero2d, out_sh.at[pl.ds(sid * NPS + b * 125, 125)])

    @pl.when(sid == 0)
    def _():
        for b in range(5):
            pltpu.sync_copy(zero1d, den_sh.at[pl.ds(b * 2000, 2000)])

    pltpu.sync_copy(s_hbm, s_l)
    pltpu.sync_copy(src_hbm.at[wid], src_l)
    pltpu.sync_copy(dst_hbm.at[wid], dst_l)
    pltpu.sync_copy(pr_hbm.at[wid], pr_l)
    plsc.subcore_barrier()

    zero16i = jnp.zeros((16,), jnp.int32)
    one16i = jnp.ones((16,), jnp.int32)

    @pl.loop(0, ROWS)
    def _(g):
        pltpu.sync_copy(z_hbm.at[src_l.at[g]], zbuf)
        for c in range(RLEN // 16):
            sl = pl.ds(c * 16, 16)
            srcv = src_l[g, sl]
            dstv = dst_l[g, sl]
            prv = pr_l[g, sl]
            s1v = plsc.load_gather(s_l, [srcv, zero16i])
            s2v = plsc.load_gather(s_l, [dstv, one16i])
            av = s1v + s2v
            ev = jnp.where(av > 0, av, av * np.float32(0.01))
            pv = jnp.exp(prv * ev)
            prow[sl, 0] = pv
            for lane in range(16):
                pb = _lane_bcast(pv, lane)
                j = c * 16 + lane
                for r in range(D // 16):
                    rsl = pl.ds(r * 16, 16)
                    zbuf[j, rsl] = zbuf[j, rsl] * pb
        pltpu.sync_copy(prow, den_sh.at[dst_l.at[g]], add=True)
        pltpu.sync_copy(zbuf, out_sh.at[dst_l.at[g]], add=True)

    plsc.subcore_barrier()

    for b in range(5):
        sl = pl.ds(sid * NPS + b * 125, 125)
        pltpu.sync_copy(out_sh.at[sl], out_hbm.at[cid].at[sl])

    @pl.when(sid == 0)
    def _():
        pltpu.sync_copy(den_sh, den_hbm.at[cid])


def _sc_edge(z, s, src3, dst3, pr3):
    mesh = plsc.VectorSubcoreMesh(core_axis_name="c", subcore_axis_name="s")
    kern = pl.kernel(
        _sc_body,
        out_type=(
            jax.ShapeDtypeStruct((NCORE, N, D), jnp.float32),
            jax.ShapeDtypeStruct((NCORE, N, 1), jnp.float32),
        ),
        mesh=mesh,
        scratch_types=[
            pltpu.VMEM((N, 2), jnp.float32),
            pltpu.VMEM((ROWS, RLEN), jnp.int32),
            pltpu.VMEM((ROWS, RLEN), jnp.int32),
            pltpu.VMEM((ROWS, RLEN), jnp.float32),
            pltpu.VMEM((RLEN, 1), jnp.float32),
            pltpu.VMEM((RLEN, D), jnp.float32),
            pltpu.VMEM((125, D), jnp.float32),
            pltpu.VMEM((2000, 1), jnp.float32),
            pltpu.VMEM_SHARED((N, D), jnp.float32),
            pltpu.VMEM_SHARED((N, 1), jnp.float32),
        ],
    )
    return kern(z, s, src3, dst3, pr3)


def _combine_body(p_ref, d_ref, o_ref):
    ps = p_ref[0] + p_ref[1]
    dsum = d_ref[0] + d_ref[1]
    safe = jnp.where(dsum > 0, dsum, np.float32(1.0))
    o_ref[...] = jnp.where(dsum > 0, ps / safe, np.float32(0.0))


def _tc_combine(out_parts, den_parts):
    blk = 1000
    return pl.pallas_call(
        _combine_body,
        grid=(N // blk,),
        in_specs=[
            pl.BlockSpec((NCORE, blk, D), lambda i: (0, i, 0)),
            pl.BlockSpec((NCORE, blk, 1), lambda i: (0, i, 0)),
        ],
        out_specs=pl.BlockSpec((blk, D), lambda i: (i, 0)),
        out_shape=jax.ShapeDtypeStruct((N, D), jnp.float32),
    )(out_parts, den_parts)


def kernel(h, edge_index, dist, fc_W, attn_W):
    src3 = edge_index[0].reshape(NW, ROWS, RLEN)
    dst3 = edge_index[1].reshape(NW, ROWS, RLEN)
    dist2d = dist.reshape(E // D, D)
    attn_T = attn_W.reshape(2, D).T
    z, s, probs = _tc_prep(h, fc_W, attn_T, dist2d)
    pr3 = probs.reshape(NW, ROWS, RLEN)
    out_parts, den_parts = _sc_edge(z, s, src3, dst3, pr3)
    return _tc_combine(out_parts, den_parts)


# speedup vs baseline: 19.3283x; 19.3283x over previous
import dataclasses
import math

import numpy as np
import jax
import jax.numpy as jnp
from jax import lax
from jax.experimental import pallas as pl
from jax.experimental.pallas import tpu as pltpu
from jax.experimental.pallas import tpu_sc as plsc

N = 10000
E = 320000
D = 128
NCORE = 2
NSUB = 16
NW = NCORE * NSUB
EPW = E // NW
ROWS = 125
RLEN = 80
NPS = N // NSUB
SEC = 25

_LOG_FACT = [float(math.lgamma(k + 1)) for k in range(10)]


def _prep_body(h_ref, w_ref, a_ref, d_ref, z_ref, s_ref, p_ref):
    z = jnp.dot(h_ref[...], w_ref[...].T, preferred_element_type=jnp.float32)
    z_ref[...] = z
    s_ref[...] = jnp.dot(z, a_ref[...], preferred_element_type=jnp.float32)
    dist = d_ref[...]
    mu = jnp.sum(dist).astype(jnp.float32) / np.float32(E)
    kv = lax.shift_right_logical(dist, 1)
    lf = jnp.zeros(dist.shape, jnp.float32)
    for j in range(1, 10):
        lf = jnp.where(kv == j, np.float32(_LOG_FACT[j]), lf)
    kf = kv.astype(jnp.float32)
    p_ref[...] = jnp.exp(kf * jnp.log(mu) - mu - lf)


def _tc_prep(h, fc_W, attn_T, dist2d):
    return pl.pallas_call(
        _prep_body,
        out_shape=(
            jax.ShapeDtypeStruct((N, D), jnp.float32),
            jax.ShapeDtypeStruct((N, 2), jnp.float32),
            jax.ShapeDtypeStruct((E // D, D), jnp.float32),
        ),
    )(h, fc_W, attn_T, dist2d)


def _lane_bcast(v, lane):
    idx = jnp.full((16, 1), lane, dtype=jnp.int32)
    dn = lax.GatherDimensionNumbers(
        offset_dims=(), collapsed_slice_dims=(0,), start_index_map=(0,))
    return lax.gather(v, idx, dn, (1,),
                      mode=lax.GatherScatterMode.PROMISE_IN_BOUNDS)


def _sc_body(z_hbm, s_hbm, src_hbm, dst_hbm, pr_hbm,
             out_hbm, den_hbm,
             s_l, src_l, dst_l, pr_l, prow, zbuf, zero1d,
             out_sh, den_sh):
    cid = lax.axis_index("c")
    sid = lax.axis_index("s")
    wid = sid * NCORE + cid

    zv16 = jnp.zeros((16,), jnp.float32)

    @pl.loop(0, RLEN)
    def _(j):
        for r in range(D // 16):
            zbuf[j, pl.ds(r * 16, 16)] = zv16

    @pl.loop(0, 2000, step=16)
    def _(i):
        zero1d[pl.ds(i, 16)] = zv16

    rbase = sid * 640

    @pl.when(sid < 15)
    def _():
        for b in range(8):
            pltpu.sync_copy(zbuf, out_sh.at[pl.ds(rbase + b * RLEN, RLEN)])

    @pl.when(sid == 15)
    def _():
        for b in range(5):
            pltpu.sync_copy(zbuf, out_sh.at[pl.ds(rbase + b * RLEN, RLEN)])

    @pl.when(sid == 0)
    def _():
        for b in range(5):
            pltpu.sync_copy(zero1d, den_sh.at[pl.ds(b * 2000, 2000)])

    pltpu.sync_copy(s_hbm, s_l)
    plsc.subcore_barrier()

    one16i = jnp.ones((16,), jnp.int32)

    @pl.loop(0, ROWS // SEC)
    def _(sec):
        ebase = wid * EPW + sec * (SEC * RLEN)
        pltpu.sync_copy(src_hbm.at[pl.ds(ebase, SEC * RLEN)], src_l)
        pltpu.sync_copy(pr_hbm.at[pl.ds(ebase, SEC * RLEN)], pr_l)

        @pl.loop(0, SEC)
        def _(g):
            pltpu.sync_copy(dst_hbm.at[pl.ds(ebase + g * RLEN, RLEN)],
                            dst_l.at[g])

        @pl.loop(0, SEC)
        def _(g):
            pltpu.sync_copy(z_hbm.at[src_l.at[pl.ds(g * RLEN, RLEN)]], zbuf)
            for c in range(RLEN // 16):
                sl = pl.ds(c * 16, 16)
                gsl = pl.ds(g * RLEN + c * 16, 16)
                srcv = src_l[gsl]
                dstv = dst_l[g, sl]
                prv = pr_l[gsl]
                s1v = plsc.load_gather(s_l, [srcv * 2])
                s2v = plsc.load_gather(s_l, [dstv * 2 + one16i])
                av = s1v + s2v
                ev = jnp.where(av > 0, av, av * np.float32(0.01))
                pv = jnp.exp(prv * ev)
                prow[sl] = pv
                for lane in range(16):
                    pb = _lane_bcast(pv, lane)
                    j = c * 16 + lane
                    for r in range(D // 16):
                        rsl = pl.ds(r * 16, 16)
                        zbuf[j, rsl] = zbuf[j, rsl] * pb
            pltpu.sync_copy(prow, den_sh.at[dst_l.at[g]], add=True)
            pltpu.sync_copy(zbuf, out_sh.at[dst_l.at[g]], add=True)

    plsc.subcore_barrier()

    @pl.when(sid < 15)
    def _():
        for b in range(8):
            sl = pl.ds(rbase + b * RLEN, RLEN)
            pltpu.sync_copy(out_sh.at[sl], out_hbm.at[cid].at[sl])

    @pl.when(sid == 15)
    def _():
        for b in range(5):
            sl = pl.ds(rbase + b * RLEN, RLEN)
            pltpu.sync_copy(out_sh.at[sl], out_hbm.at[cid].at[sl])

    @pl.when(sid == 0)
    def _():
        pltpu.sync_copy(den_sh, den_hbm.at[cid])


def _sc_edge(z, s, src3, dst3, pr3):
    mesh = plsc.VectorSubcoreMesh(core_axis_name="c", subcore_axis_name="s")
    cp = pltpu.CompilerParams()
    if "needs_layout_passes" in pltpu.CompilerParams.__dataclass_fields__:
        cp = dataclasses.replace(cp, needs_layout_passes=False)
    kern = pl.kernel(
        _sc_body,
        compiler_params=cp,
        out_type=(
            jax.ShapeDtypeStruct((NCORE, N, D), jnp.float32),
            jax.ShapeDtypeStruct((NCORE, N), jnp.float32),
        ),
        mesh=mesh,
        scratch_types=[
            pltpu.VMEM((2 * N,), jnp.float32),
            pltpu.VMEM((SEC * RLEN,), jnp.int32),
            pltpu.VMEM((SEC, RLEN), jnp.int32),
            pltpu.VMEM((SEC * RLEN,), jnp.float32),
            pltpu.VMEM((RLEN,), jnp.float32),
            pltpu.VMEM((RLEN, D), jnp.float32),
            pltpu.VMEM((2000,), jnp.float32),
            pltpu.VMEM_SHARED((N, D), jnp.float32),
            pltpu.VMEM_SHARED((N,), jnp.float32),
        ],
    )
    return kern(z, s, src3, dst3, pr3)


def _combine_body(p_ref, d_ref, o_ref):
    ps = p_ref[0] + p_ref[1]
    dsum = d_ref[0] + d_ref[1]
    safe = jnp.where(dsum > 0, dsum, np.float32(1.0))
    o_ref[...] = jnp.where(dsum > 0, ps / safe, np.float32(0.0))


def _tc_combine(out_parts, den_parts):
    blk = 1000
    return pl.pallas_call(
        _combine_body,
        grid=(N // blk,),
        in_specs=[
            pl.BlockSpec((NCORE, blk, D), lambda i: (0, i, 0)),
            pl.BlockSpec((NCORE, blk, 1), lambda i: (0, i, 0)),
        ],
        out_specs=pl.BlockSpec((blk, D), lambda i: (i, 0)),
        out_shape=jax.ShapeDtypeStruct((N, D), jnp.float32),
    )(out_parts, den_parts)


def kernel(h, edge_index, dist, fc_W, attn_W):
    src = edge_index[0]
    dst = edge_index[1]
    dist2d = dist.reshape(E // D, D)
    attn_T = attn_W.reshape(2, D).T
    z, s, probs = _tc_prep(h, fc_W, attn_T, dist2d)
    out_parts, den_parts = _sc_edge(z, s.reshape(2 * N), src, dst,
                                    probs.reshape(E))
    return _tc_combine(out_parts, den_parts.reshape(NCORE, N, 1))


# speedup vs baseline: 21.5895x; 1.1170x over previous
import dataclasses
import math

import numpy as np
import jax
import jax.numpy as jnp
from jax import lax
from jax.experimental import pallas as pl
from jax.experimental.pallas import tpu as pltpu
from jax.experimental.pallas import tpu_sc as plsc

N = 10000
E = 320000
D = 128
NCORE = 2
NSUB = 16
NW = NCORE * NSUB
EPW = E // NW
ROWS = 125
RLEN = 80
NPS = N // NSUB
SEC = 25

_LOG_FACT = [float(math.lgamma(k + 1)) for k in range(10)]


def _prep_body(h_ref, w_ref, a_ref, d_ref, z_ref, s_ref, p_ref):
    z = jnp.dot(h_ref[...], w_ref[...].T, preferred_element_type=jnp.float32)
    z_ref[...] = z
    s_ref[...] = jnp.dot(z, a_ref[...], preferred_element_type=jnp.float32)
    dist = d_ref[...]
    mu = jnp.sum(dist).astype(jnp.float32) / np.float32(E)
    kv = lax.shift_right_logical(dist, 1)
    lf = jnp.zeros(dist.shape, jnp.float32)
    for j in range(1, 10):
        lf = jnp.where(kv == j, np.float32(_LOG_FACT[j]), lf)
    kf = kv.astype(jnp.float32)
    p_ref[...] = jnp.exp(kf * jnp.log(mu) - mu - lf)


def _tc_prep(h, fc_W, attn_T, dist2d):
    return pl.pallas_call(
        _prep_body,
        out_shape=(
            jax.ShapeDtypeStruct((N, D), jnp.float32),
            jax.ShapeDtypeStruct((N, 2), jnp.float32),
            jax.ShapeDtypeStruct((E // D, D), jnp.float32),
        ),
    )(h, fc_W, attn_T, dist2d)


def _lane_bcast(v, lane):
    idx = jnp.full((16, 1), lane, dtype=jnp.int32)
    dn = lax.GatherDimensionNumbers(
        offset_dims=(), collapsed_slice_dims=(0,), start_index_map=(0,))
    return lax.gather(v, idx, dn, (1,),
                      mode=lax.GatherScatterMode.PROMISE_IN_BOUNDS)


def _sc_body(z_hbm, s_hbm, src_hbm, dst_hbm, pr_hbm,
             out_hbm, den_hbm,
             s_l, src_l, dst_l, pr_l, prow, zbufa, zbufb,
             sga, sgb, ssa, ssb,
             out_sh, den_sh):
    cid = lax.axis_index("c")
    sid = lax.axis_index("s")
    wid = sid * NCORE + cid

    zv16 = jnp.zeros((16,), jnp.float32)

    @pl.loop(0, RLEN)
    def _(j):
        for r in range(D // 16):
            zbufa[j, pl.ds(r * 16, 16)] = zv16

    for r in range(RLEN // 16):
        prow[pl.ds(r * 16, 16)] = zv16

    rbase = sid * 640

    @pl.when(sid < 15)
    def _():
        for b in range(8):
            pltpu.sync_copy(zbufa, out_sh.at[pl.ds(rbase + b * RLEN, RLEN)])

    @pl.when(sid == 15)
    def _():
        for b in range(5):
            pltpu.sync_copy(zbufa, out_sh.at[pl.ds(rbase + b * RLEN, RLEN)])

    @pl.when(sid == 0)
    def _():
        @pl.loop(0, N, step=RLEN)
        def _(i):
            pltpu.sync_copy(prow, den_sh.at[pl.ds(i, RLEN)])

    pltpu.sync_copy(s_hbm, s_l)
    plsc.subcore_barrier()

    one16i = jnp.ones((16,), jnp.int32)

    def gstart(g, zb, sem):
        pltpu.async_copy(z_hbm.at[src_l.at[pl.ds(g * RLEN, RLEN)]], zb, sem)

    def gwait(zb, sem):
        pltpu.make_async_copy(z_hbm.at[src_l.at[pl.ds(0, RLEN)]], zb,
                              sem).wait()

    def sstart(g, zb, sem):
        pltpu.async_copy(zb, out_sh.at[dst_l.at[g]], sem, add=True)

    def swait(zb, sem):
        pltpu.make_async_copy(zb, out_sh.at[dst_l.at[0]], sem).wait()

    def compute_group(g, zb):
        for c in range(RLEN // 16):
            sl = pl.ds(c * 16, 16)
            gsl = pl.ds(g * RLEN + c * 16, 16)
            srcv = src_l[gsl]
            dstv = dst_l[g, sl]
            prv = pr_l[gsl]
            s1v = plsc.load_gather(s_l, [srcv * 2])
            s2v = plsc.load_gather(s_l, [dstv * 2 + one16i])
            av = s1v + s2v
            ev = jnp.where(av > 0, av, av * np.float32(0.01))
            pv = jnp.exp(prv * ev)
            prow[sl] = pv
            for lane in range(16):
                pb = _lane_bcast(pv, lane)
                j = c * 16 + lane
                for r in range(D // 16):
                    rsl = pl.ds(r * 16, 16)
                    zb[j, rsl] = zb[j, rsl] * pb
        pltpu.sync_copy(prow, den_sh.at[dst_l.at[g]], add=True)

    @pl.loop(0, ROWS // SEC)
    def _(sec):
        ebase = wid * EPW + sec * (SEC * RLEN)
        pltpu.sync_copy(src_hbm.at[pl.ds(ebase, SEC * RLEN)], src_l)
        pltpu.sync_copy(pr_hbm.at[pl.ds(ebase, SEC * RLEN)], pr_l)

        @pl.loop(0, SEC)
        def _(g):
            pltpu.sync_copy(dst_hbm.at[pl.ds(ebase + g * RLEN, RLEN)],
                            dst_l.at[g])

        gstart(0, zbufa, sga)
        gstart(1, zbufb, sgb)

        @pl.loop(0, SEC - 1, step=2)
        def _(g):
            gwait(zbufa, sga)
            compute_group(g, zbufa)
            sstart(g, zbufa, ssa)
            gwait(zbufb, sgb)
            compute_group(g + 1, zbufb)
            sstart(g + 1, zbufb, ssb)
            swait(zbufa, ssa)
            gstart(g + 2, zbufa, sga)

            @pl.when(g + 3 < SEC)
            def _():
                swait(zbufb, ssb)
                gstart(g + 3, zbufb, sgb)

        gwait(zbufa, sga)
        compute_group(SEC - 1, zbufa)
        pltpu.sync_copy(zbufa, out_sh.at[dst_l.at[SEC - 1]], add=True)
        swait(zbufb, ssb)

    plsc.subcore_barrier()

    @pl.when(sid < 15)
    def _():
        for b in range(8):
            sl = pl.ds(rbase + b * RLEN, RLEN)
            pltpu.sync_copy(out_sh.at[sl], out_hbm.at[cid].at[sl])

    @pl.when(sid == 15)
    def _():
        for b in range(5):
            sl = pl.ds(rbase + b * RLEN, RLEN)
            pltpu.sync_copy(out_sh.at[sl], out_hbm.at[cid].at[sl])

    @pl.when(sid == 0)
    def _():
        pltpu.sync_copy(den_sh, den_hbm.at[cid])


def _sc_edge(z, s, src3, dst3, pr3):
    mesh = plsc.VectorSubcoreMesh(core_axis_name="c", subcore_axis_name="s")
    cp = pltpu.CompilerParams()
    if "needs_layout_passes" in pltpu.CompilerParams.__dataclass_fields__:
        cp = dataclasses.replace(cp, needs_layout_passes=False)
    kern = pl.kernel(
        _sc_body,
        compiler_params=cp,
        out_type=(
            jax.ShapeDtypeStruct((NCORE, N, D), jnp.float32),
            jax.ShapeDtypeStruct((NCORE, N), jnp.float32),
        ),
        mesh=mesh,
        scratch_types=[
            pltpu.VMEM((2 * N,), jnp.float32),
            pltpu.VMEM((SEC * RLEN,), jnp.int32),
            pltpu.VMEM((SEC, RLEN), jnp.int32),
            pltpu.VMEM((SEC * RLEN,), jnp.float32),
            pltpu.VMEM((RLEN,), jnp.float32),
            pltpu.VMEM((RLEN, D), jnp.float32),
            pltpu.VMEM((RLEN, D), jnp.float32),
            pltpu.SemaphoreType.DMA,
            pltpu.SemaphoreType.DMA,
            pltpu.SemaphoreType.DMA,
            pltpu.SemaphoreType.DMA,
            pltpu.VMEM_SHARED((N, D), jnp.float32),
            pltpu.VMEM_SHARED((N,), jnp.float32),
        ],
    )
    return kern(z, s, src3, dst3, pr3)


def _combine_body(p_ref, d_ref, o_ref):
    ps = p_ref[0] + p_ref[1]
    dsum = d_ref[0] + d_ref[1]
    safe = jnp.where(dsum > 0, dsum, np.float32(1.0))
    o_ref[...] = jnp.where(dsum > 0, ps / safe, np.float32(0.0))


def _tc_combine(out_parts, den_parts):
    blk = 1000
    return pl.pallas_call(
        _combine_body,
        grid=(N // blk,),
        in_specs=[
            pl.BlockSpec((NCORE, blk, D), lambda i: (0, i, 0)),
            pl.BlockSpec((NCORE, blk, 1), lambda i: (0, i, 0)),
        ],
        out_specs=pl.BlockSpec((blk, D), lambda i: (i, 0)),
        out_shape=jax.ShapeDtypeStruct((N, D), jnp.float32),
    )(out_parts, den_parts)


def kernel(h, edge_index, dist, fc_W, attn_W):
    src = edge_index[0]
    dst = edge_index[1]
    dist2d = dist.reshape(E // D, D)
    attn_T = attn_W.reshape(2, D).T
    z, s, probs = _tc_prep(h, fc_W, attn_T, dist2d)
    out_parts, den_parts = _sc_edge(z, s.reshape(2 * N), src, dst,
                                    probs.reshape(E))
    return _tc_combine(out_parts, den_parts.reshape(NCORE, N, 1))


# speedup vs baseline: 24.0699x; 1.1149x over previous
import dataclasses
import math

import numpy as np
import jax
import jax.numpy as jnp
from jax import lax
from jax.experimental import pallas as pl
from jax.experimental.pallas import tpu as pltpu
from jax.experimental.pallas import tpu_sc as plsc

N = 10000
E = 320000
D = 128
NCORE = 2
NSUB = 16
NW = NCORE * NSUB
EPW = E // NW
ROWS = 125
RLEN = 80
NPS = N // NSUB
SEC = 25

_LOG_FACT = [float(math.lgamma(k + 1)) for k in range(10)]


def _prep_body(h_ref, w_ref, a_ref, d_ref, z_ref, s_ref, p_ref):
    z = jnp.dot(h_ref[...], w_ref[...].T, preferred_element_type=jnp.float32)
    z_ref[...] = z
    s_ref[...] = jnp.dot(z, a_ref[...], preferred_element_type=jnp.float32)
    dist = d_ref[...]
    mu = jnp.sum(dist).astype(jnp.float32) / np.float32(E)
    kv = lax.shift_right_logical(dist, 1)
    lf = jnp.zeros(dist.shape, jnp.float32)
    for j in range(1, 10):
        lf = jnp.where(kv == j, np.float32(_LOG_FACT[j]), lf)
    kf = kv.astype(jnp.float32)
    p_ref[...] = jnp.exp(kf * jnp.log(mu) - mu - lf)


def _tc_prep(h, fc_W, attn_T, dist2d):
    return pl.pallas_call(
        _prep_body,
        out_shape=(
            jax.ShapeDtypeStruct((N, D), jnp.float32),
            jax.ShapeDtypeStruct((N, 2), jnp.float32),
            jax.ShapeDtypeStruct((E // D, D), jnp.float32),
        ),
    )(h, fc_W, attn_T, dist2d)


def _lane_bcast(v, lane):
    idx = jnp.full((16, 1), lane, dtype=jnp.int32)
    dn = lax.GatherDimensionNumbers(
        offset_dims=(), collapsed_slice_dims=(0,), start_index_map=(0,))
    return lax.gather(v, idx, dn, (1,),
                      mode=lax.GatherScatterMode.PROMISE_IN_BOUNDS)


def _sc_body(z_hbm, s_hbm, src_hbm, dst_hbm, pr_hbm,
             out_hbm, den_hbm,
             s_l, src_l, dst_flat, dst_l, pr_l, prow,
             zbufa, zbufb,
             sga, sgb, ssa, ssb,
             out_sh, den_sh):
    cid = lax.axis_index("c")
    sid = lax.axis_index("s")
    wid = sid * NCORE + cid

    zv16 = jnp.zeros((16,), jnp.float32)

    @pl.loop(0, RLEN)
    def _(j):
        for r in range(D // 16):
            zbufa[j, pl.ds(r * 16, 16)] = zv16

    for r in range(RLEN // 16):
        prow[pl.ds(r * 16, 16)] = zv16

    rbase = sid * 640

    @pl.when(sid < 15)
    def _():
        for b in range(8):
            pltpu.sync_copy(zbufa, out_sh.at[pl.ds(rbase + b * RLEN, RLEN)])

    @pl.when(sid == 15)
    def _():
        for b in range(5):
            pltpu.sync_copy(zbufa, out_sh.at[pl.ds(rbase + b * RLEN, RLEN)])

    @pl.when(sid == 0)
    def _():
        @pl.loop(0, N, step=RLEN)
        def _(i):
            pltpu.sync_copy(prow, den_sh.at[pl.ds(i, RLEN)])

    pltpu.sync_copy(s_hbm, s_l)
    plsc.subcore_barrier()

    def gstart(g, zb, sem):
        pltpu.async_copy(z_hbm.at[src_l.at[pl.ds(g * RLEN, RLEN)]], zb, sem)

    def gwait(zb, sem):
        pltpu.make_async_copy(z_hbm.at[src_l.at[pl.ds(0, RLEN)]], zb,
                              sem).wait()

    def sstart(g, zb, sem):
        pltpu.async_copy(zb, out_sh.at[dst_l.at[g]], sem, add=True)

    def swait(zb, sem):
        pltpu.make_async_copy(zb, out_sh.at[dst_l.at[0]], sem).wait()

    def compute_group(g, zb):
        for c in range(RLEN // 16):
            sl = pl.ds(c * 16, 16)
            gsl = pl.ds(g * RLEN + c * 16, 16)
            srcv = src_l[gsl]
            dstv = dst_l[g, sl]
            prv = pr_l[gsl]
            w1 = plsc.load_gather(s_l, [srcv])
            w2 = plsc.load_gather(s_l, [dstv])
            s1v = plsc.bitcast(lax.shift_left(w1, 16), jnp.float32)
            s2v = plsc.bitcast(w2 & np.int32(-65536), jnp.float32)
            av = s1v + s2v
            ev = jnp.where(av > 0, av, av * np.float32(0.01))
            pv = jnp.exp(prv * ev)
            prow[sl] = pv
            for lane in range(16):
                pb = _lane_bcast(pv, lane)
                j = c * 16 + lane
                for r in range(D // 16):
                    rsl = pl.ds(r * 16, 16)
                    zb[j, rsl] = zb[j, rsl] * pb
        pltpu.sync_copy(prow, den_sh.at[dst_l.at[g]], add=True)

    @pl.loop(0, ROWS // SEC)
    def _(sec):
        ebase = wid * EPW + sec * (SEC * RLEN)
        pltpu.sync_copy(src_hbm.at[pl.ds(ebase, SEC * RLEN)], src_l)
        pltpu.sync_copy(pr_hbm.at[pl.ds(ebase, SEC * RLEN)], pr_l)
        pltpu.sync_copy(dst_hbm.at[pl.ds(ebase, SEC * RLEN)], dst_flat)

        @pl.loop(0, SEC)
        def _(g):
            for c in range(RLEN // 16):
                dst_l[g, pl.ds(c * 16, 16)] = \
                    dst_flat[pl.ds(g * RLEN + c * 16, 16)]

        gstart(0, zbufa, sga)
        gstart(1, zbufb, sgb)

        @pl.loop(0, SEC - 1, step=2)
        def _(g):
            gwait(zbufa, sga)
            compute_group(g, zbufa)
            sstart(g, zbufa, ssa)
            gwait(zbufb, sgb)
            compute_group(g + 1, zbufb)
            sstart(g + 1, zbufb, ssb)
            swait(zbufa, ssa)
            gstart(g + 2, zbufa, sga)

            @pl.when(g + 3 < SEC)
            def _():
                swait(zbufb, ssb)
                gstart(g + 3, zbufb, sgb)

        gwait(zbufa, sga)
        compute_group(SEC - 1, zbufa)
        pltpu.sync_copy(zbufa, out_sh.at[dst_l.at[SEC - 1]], add=True)
        swait(zbufb, ssb)

    plsc.subcore_barrier()

    @pl.when(sid < 15)
    def _():
        for b in range(8):
            sl = pl.ds(rbase + b * RLEN, RLEN)
            pltpu.sync_copy(out_sh.at[sl], out_hbm.at[cid].at[sl])

    @pl.when(sid == 15)
    def _():
        for b in range(5):
            sl = pl.ds(rbase + b * RLEN, RLEN)
            pltpu.sync_copy(out_sh.at[sl], out_hbm.at[cid].at[sl])

    @pl.when(sid == 0)
    def _():
        pltpu.sync_copy(den_sh, den_hbm.at[cid])


def _sc_edge(z, s, src3, dst3, pr3):
    mesh = plsc.VectorSubcoreMesh(core_axis_name="c", subcore_axis_name="s")
    cp = pltpu.CompilerParams()
    if "needs_layout_passes" in pltpu.CompilerParams.__dataclass_fields__:
        cp = dataclasses.replace(cp, needs_layout_passes=False)
    kern = pl.kernel(
        _sc_body,
        compiler_params=cp,
        out_type=(
            jax.ShapeDtypeStruct((NCORE, N, D), jnp.float32),
            jax.ShapeDtypeStruct((NCORE, N), jnp.float32),
        ),
        mesh=mesh,
        scratch_types=[
            pltpu.VMEM((N,), jnp.int32),
            pltpu.VMEM((SEC * RLEN,), jnp.int32),
            pltpu.VMEM((SEC * RLEN,), jnp.int32),
            pltpu.VMEM((SEC, RLEN), jnp.int32),
            pltpu.VMEM((SEC * RLEN,), jnp.float32),
            pltpu.VMEM((RLEN,), jnp.float32),
            pltpu.VMEM((RLEN, D), jnp.float32),
            pltpu.VMEM((RLEN, D), jnp.float32),
            pltpu.SemaphoreType.DMA,
            pltpu.SemaphoreType.DMA,
            pltpu.SemaphoreType.DMA,
            pltpu.SemaphoreType.DMA,
            pltpu.VMEM_SHARED((N, D), jnp.float32),
            pltpu.VMEM_SHARED((N,), jnp.float32),
        ],
    )
    return kern(z, s, src3, dst3, pr3)


def _combine_body(p_ref, d_ref, o_ref):
    ps = p_ref[0] + p_ref[1]
    dsum = d_ref[0] + d_ref[1]
    safe = jnp.where(dsum > 0, dsum, np.float32(1.0))
    o_ref[...] = jnp.where(dsum > 0, ps / safe, np.float32(0.0))


def _tc_combine(out_parts, den_parts):
    blk = 1000
    return pl.pallas_call(
        _combine_body,
        grid=(N // blk,),
        in_specs=[
            pl.BlockSpec((NCORE, blk, D), lambda i: (0, i, 0)),
            pl.BlockSpec((NCORE, blk, 1), lambda i: (0, i, 0)),
        ],
        out_specs=pl.BlockSpec((blk, D), lambda i: (i, 0)),
        out_shape=jax.ShapeDtypeStruct((N, D), jnp.float32),
    )(out_parts, den_parts)


def kernel(h, edge_index, dist, fc_W, attn_W):
    src = edge_index[0]
    dst = edge_index[1]
    dist2d = dist.reshape(E // D, D)
    attn_T = attn_W.reshape(2, D).T
    z, s, probs = _tc_prep(h, fc_W, attn_T, dist2d)
    s_pack = lax.bitcast_convert_type(s.astype(jnp.bfloat16), jnp.int32)
    out_parts, den_parts = _sc_edge(z, s_pack, src, dst, probs.reshape(E))
    return _tc_combine(out_parts, den_parts.reshape(NCORE, N, 1))


# speedup vs baseline: 24.5793x; 1.0212x over previous
import dataclasses
import math

import numpy as np
import jax
import jax.numpy as jnp
from jax import lax
from jax.experimental import pallas as pl
from jax.experimental.pallas import tpu as pltpu
from jax.experimental.pallas import tpu_sc as plsc

N = 10000
E = 320000
D = 128
NCORE = 2
NSUB = 16
NW = NCORE * NSUB
EPW = E // NW
ROWS = 125
RLEN = 80
NPS = N // NSUB
SEC = 25

_LOG_FACT = [float(math.lgamma(k + 1)) for k in range(10)]


def _prep_body(h_ref, w_ref, a_ref, d_ref, z_ref, s_ref, p_ref):
    z = jnp.dot(h_ref[...], w_ref[...].T, preferred_element_type=jnp.float32)
    z_ref[...] = z
    s_ref[...] = jnp.dot(z, a_ref[...], preferred_element_type=jnp.float32)
    dist = d_ref[...]
    mu = jnp.sum(dist).astype(jnp.float32) / np.float32(E)
    kv = lax.shift_right_logical(dist, 1)
    lf = jnp.zeros(dist.shape, jnp.float32)
    for j in range(1, 10):
        lf = jnp.where(kv == j, np.float32(_LOG_FACT[j]), lf)
    kf = kv.astype(jnp.float32)
    p_ref[...] = jnp.exp(kf * jnp.log(mu) - mu - lf)


def _tc_prep(h, fc_W, attn_T, dist2d):
    return pl.pallas_call(
        _prep_body,
        out_shape=(
            jax.ShapeDtypeStruct((N, D), jnp.float32),
            jax.ShapeDtypeStruct((N, 2), jnp.float32),
            jax.ShapeDtypeStruct((E // D, D), jnp.float32),
        ),
    )(h, fc_W, attn_T, dist2d)


def _lane_bcast(v, lane):
    idx = jnp.full((16, 1), lane, dtype=jnp.int32)
    dn = lax.GatherDimensionNumbers(
        offset_dims=(), collapsed_slice_dims=(0,), start_index_map=(0,))
    return lax.gather(v, idx, dn, (1,),
                      mode=lax.GatherScatterMode.PROMISE_IN_BOUNDS)


def _sc_body(z_hbm, s_hbm, src_hbm, dst_hbm, pr_hbm,
             out_hbm, den_hbm,
             s_l, src_l, dst_flat, dst_l, pr_l, prowa, prowb, zero1d,
             zbufa, zbufb,
             sga, sgb, ssa, ssb, sda, sdb,
             out_sh, den_sh):
    cid = lax.axis_index("c")
    sid = lax.axis_index("s")
    wid = sid * NCORE + cid

    zv16 = jnp.zeros((16,), jnp.float32)

    @pl.loop(0, RLEN)
    def _(j):
        for r in range(D // 16):
            zbufa[j, pl.ds(r * 16, 16)] = zv16

    @pl.loop(0, 2000, step=16)
    def _(i):
        zero1d[pl.ds(i, 16)] = zv16

    rbase = sid * 640

    @pl.when(sid < 15)
    def _():
        for b in range(8):
            pltpu.sync_copy(zbufa, out_sh.at[pl.ds(rbase + b * RLEN, RLEN)])

    @pl.when(sid == 15)
    def _():
        for b in range(5):
            pltpu.sync_copy(zbufa, out_sh.at[pl.ds(rbase + b * RLEN, RLEN)])

    @pl.when(sid == 0)
    def _():
        for b in range(5):
            pltpu.sync_copy(zero1d, den_sh.at[pl.ds(b * 2000, 2000)])

    pltpu.sync_copy(s_hbm, s_l)
    plsc.subcore_barrier()

    def gstart(g, zb, sem):
        pltpu.async_copy(z_hbm.at[src_l.at[pl.ds(g * RLEN, RLEN)]], zb, sem)

    def gwait(zb, sem):
        pltpu.make_async_copy(z_hbm.at[src_l.at[pl.ds(0, RLEN)]], zb,
                              sem).wait()

    def sstart(g, zb, sem):
        pltpu.async_copy(zb, out_sh.at[dst_l.at[g]], sem, add=True)

    def swait(zb, sem):
        pltpu.make_async_copy(zb, out_sh.at[dst_l.at[0]], sem).wait()

    def dstart(g, pw, sem):
        pltpu.async_copy(pw, den_sh.at[dst_l.at[g]], sem, add=True)

    def dwait(pw, sem):
        pltpu.make_async_copy(pw, den_sh.at[dst_l.at[0]], sem).wait()

    def compute_group(g, zb, pw):
        for c in range(RLEN // 16):
            sl = pl.ds(c * 16, 16)
            gsl = pl.ds(g * RLEN + c * 16, 16)
            srcv = src_l[gsl]
            dstv = dst_l[g, sl]
            prv = pr_l[gsl]
            w1 = plsc.load_gather(s_l, [srcv])
            w2 = plsc.load_gather(s_l, [dstv])
            s1v = plsc.bitcast(lax.shift_left(w1, 16), jnp.float32)
            s2v = plsc.bitcast(w2 & np.int32(-65536), jnp.float32)
            av = s1v + s2v
            ev = jnp.where(av > 0, av, av * np.float32(0.01))
            pv = jnp.exp(prv * ev)
            pw[sl] = pv
            for lane in range(16):
                pb = _lane_bcast(pv, lane)
                j = c * 16 + lane
                for r in range(D // 16):
                    rsl = pl.ds(r * 16, 16)
                    zb[j, rsl] = zb[j, rsl] * pb

    @pl.loop(0, ROWS // SEC)
    def _(sec):
        ebase = wid * EPW + sec * (SEC * RLEN)
        pltpu.sync_copy(src_hbm.at[pl.ds(ebase, SEC * RLEN)], src_l)
        pltpu.sync_copy(pr_hbm.at[pl.ds(ebase, SEC * RLEN)], pr_l)
        pltpu.sync_copy(dst_hbm.at[pl.ds(ebase, SEC * RLEN)], dst_flat)

        @pl.loop(0, SEC)
        def _(g):
            for c in range(RLEN // 16):
                dst_l[g, pl.ds(c * 16, 16)] = \
                    dst_flat[pl.ds(g * RLEN + c * 16, 16)]

        gstart(0, zbufa, sga)
        gstart(1, zbufb, sgb)

        @pl.loop(0, SEC - 1, step=2)
        def _(g):
            gwait(zbufa, sga)
            compute_group(g, zbufa, prowa)
            sstart(g, zbufa, ssa)
            dstart(g, prowa, sda)
            gwait(zbufb, sgb)
            compute_group(g + 1, zbufb, prowb)
            sstart(g + 1, zbufb, ssb)
            dstart(g + 1, prowb, sdb)
            swait(zbufa, ssa)
            gstart(g + 2, zbufa, sga)
            dwait(prowa, sda)

            @pl.when(g + 3 < SEC)
            def _():
                swait(zbufb, ssb)
                gstart(g + 3, zbufb, sgb)
                dwait(prowb, sdb)

        gwait(zbufa, sga)
        compute_group(SEC - 1, zbufa, prowa)
        pltpu.sync_copy(zbufa, out_sh.at[dst_l.at[SEC - 1]], add=True)
        pltpu.sync_copy(prowa, den_sh.at[dst_l.at[SEC - 1]], add=True)
        swait(zbufb, ssb)
        dwait(prowb, sdb)

    plsc.subcore_barrier()

    @pl.when(sid < 15)
    def _():
        for b in range(8):
            sl = pl.ds(rbase + b * RLEN, RLEN)
            pltpu.sync_copy(out_sh.at[sl], out_hbm.at[cid].at[sl])

    @pl.when(sid == 15)
    def _():
        for b in range(5):
            sl = pl.ds(rbase + b * RLEN, RLEN)
            pltpu.sync_copy(out_sh.at[sl], out_hbm.at[cid].at[sl])

    @pl.when(sid == 0)
    def _():
        pltpu.sync_copy(den_sh, den_hbm.at[cid])


def _sc_edge(z, s, src3, dst3, pr3):
    mesh = plsc.VectorSubcoreMesh(core_axis_name="c", subcore_axis_name="s")
    cp = pltpu.CompilerParams()
    if "needs_layout_passes" in pltpu.CompilerParams.__dataclass_fields__:
        cp = dataclasses.replace(cp, needs_layout_passes=False)
    kern = pl.kernel(
        _sc_body,
        compiler_params=cp,
        out_type=(
            jax.ShapeDtypeStruct((NCORE, N, D), jnp.float32),
            jax.ShapeDtypeStruct((NCORE, N), jnp.float32),
        ),
        mesh=mesh,
        scratch_types=[
            pltpu.VMEM((N,), jnp.int32),
            pltpu.VMEM((SEC * RLEN,), jnp.int32),
            pltpu.VMEM((SEC * RLEN,), jnp.int32),
            pltpu.VMEM((SEC, RLEN), jnp.int32),
            pltpu.VMEM((SEC * RLEN,), jnp.float32),
            pltpu.VMEM((RLEN,), jnp.float32),
            pltpu.VMEM((RLEN,), jnp.float32),
            pltpu.VMEM((2000,), jnp.float32),
            pltpu.VMEM((RLEN, D), jnp.float32),
            pltpu.VMEM((RLEN, D), jnp.float32),
            pltpu.SemaphoreType.DMA,
            pltpu.SemaphoreType.DMA,
            pltpu.SemaphoreType.DMA,
            pltpu.SemaphoreType.DMA,
            pltpu.SemaphoreType.DMA,
            pltpu.SemaphoreType.DMA,
            pltpu.VMEM_SHARED((N, D), jnp.float32),
            pltpu.VMEM_SHARED((N,), jnp.float32),
        ],
    )
    return kern(z, s, src3, dst3, pr3)


def _combine_body(p_ref, d_ref, o_ref):
    ps = p_ref[0] + p_ref[1]
    dsum = d_ref[0] + d_ref[1]
    safe = jnp.where(dsum > 0, dsum, np.float32(1.0))
    o_ref[...] = jnp.where(dsum > 0, ps / safe, np.float32(0.0))


def _tc_combine(out_parts, den_parts):
    blk = 1000
    return pl.pallas_call(
        _combine_body,
        grid=(N // blk,),
        in_specs=[
            pl.BlockSpec((NCORE, blk, D), lambda i: (0, i, 0)),
            pl.BlockSpec((NCORE, blk, 1), lambda i: (0, i, 0)),
        ],
        out_specs=pl.BlockSpec((blk, D), lambda i: (i, 0)),
        out_shape=jax.ShapeDtypeStruct((N, D), jnp.float32),
    )(out_parts, den_parts)


def kernel(h, edge_index, dist, fc_W, attn_W):
    src = edge_index[0]
    dst = edge_index[1]
    dist2d = dist.reshape(E // D, D)
    attn_T = attn_W.reshape(2, D).T
    z, s, probs = _tc_prep(h, fc_W, attn_T, dist2d)
    s_pack = lax.bitcast_convert_type(s.astype(jnp.bfloat16), jnp.int32)
    out_parts, den_parts = _sc_edge(z, s_pack, src, dst, probs.reshape(E))
    return _tc_combine(out_parts, den_parts.reshape(NCORE, N, 1))


# speedup vs baseline: 27.3553x; 1.1129x over previous
import dataclasses
import math

import numpy as np
import jax
import jax.numpy as jnp
from jax import lax
from jax.experimental import pallas as pl
from jax.experimental.pallas import tpu as pltpu
from jax.experimental.pallas import tpu_sc as plsc

N = 10000
E = 320000
D = 128
NCORE = 2
NSUB = 16
NW = NCORE * NSUB
EPW = E // NW
ROWS = 125
RLEN = 80
NPS = N // NSUB
SEC = 25

_LOG_FACT = [float(math.lgamma(k + 1)) for k in range(10)]


def _prep_body(h_ref, w_ref, a_ref, d_ref, z_ref, s_ref, p_ref):
    z = jnp.dot(h_ref[...], w_ref[...].T, preferred_element_type=jnp.float32)
    z_ref[...] = z
    s_ref[...] = jnp.dot(z, a_ref[...], preferred_element_type=jnp.float32)
    dist = d_ref[...]
    mu = jnp.sum(dist).astype(jnp.float32) / np.float32(E)
    kv = lax.shift_right_logical(dist, 1)
    lf = jnp.zeros(dist.shape, jnp.float32)
    for j in range(1, 10):
        lf = jnp.where(kv == j, np.float32(_LOG_FACT[j]), lf)
    kf = kv.astype(jnp.float32)
    p_ref[...] = jnp.exp(kf * jnp.log(mu) - mu - lf)


def _tc_prep(h, fc_W, attn_T, dist2d):
    return pl.pallas_call(
        _prep_body,
        out_shape=(
            jax.ShapeDtypeStruct((N, D), jnp.float32),
            jax.ShapeDtypeStruct((N, 2), jnp.float32),
            jax.ShapeDtypeStruct((E // D, D), jnp.float32),
        ),
    )(h, fc_W, attn_T, dist2d)


def _lane_bcast(v, lane):
    idx = jnp.full((16, 1), lane, dtype=jnp.int32)
    dn = lax.GatherDimensionNumbers(
        offset_dims=(), collapsed_slice_dims=(0,), start_index_map=(0,))
    return lax.gather(v, idx, dn, (1,),
                      mode=lax.GatherScatterMode.PROMISE_IN_BOUNDS)


def _sc_body(z_hbm, s_hbm, src_hbm, dst_hbm, pr_hbm,
             out_hbm, den_hbm,
             s_l, src_l, dst_l, pr_l, prowa, prowb, prowc,
             zbufa, zbufb, zbufc,
             sga, sgb, sgc, ssa, ssb, ssc, sda, sdb, sdc,
             out_sh, den_sh):
    cid = lax.axis_index("c")
    sid = lax.axis_index("s")
    wid = sid * NCORE + cid

    zv16 = jnp.zeros((16,), jnp.float32)

    @pl.loop(0, RLEN)
    def _(j):
        for r in range(D // 16):
            zbufa[j, pl.ds(r * 16, 16)] = zv16

    for pw in (prowa, prowb, prowc):
        for r in range(RLEN // 16):
            pw[pl.ds(r * 16, 16)] = zv16

    rbase = sid * 640

    @pl.when(sid < 15)
    def _():
        for b in range(8):
            pltpu.sync_copy(zbufa, out_sh.at[pl.ds(rbase + b * RLEN, RLEN)])
            pltpu.sync_copy(prowa, den_sh.at[pl.ds(rbase + b * RLEN, RLEN)])

    @pl.when(sid == 15)
    def _():
        for b in range(5):
            pltpu.sync_copy(zbufa, out_sh.at[pl.ds(rbase + b * RLEN, RLEN)])
            pltpu.sync_copy(prowa, den_sh.at[pl.ds(rbase + b * RLEN, RLEN)])

    pltpu.sync_copy(s_hbm, s_l)
    plsc.subcore_barrier()

    def gstart(g, zb, sem):
        pltpu.async_copy(z_hbm.at[src_l.at[pl.ds(g * RLEN, RLEN)]], zb, sem)

    def gwait(zb, sem):
        pltpu.make_async_copy(z_hbm.at[src_l.at[pl.ds(0, RLEN)]], zb,
                              sem).wait()

    def sstart(g, zb, sem):
        pltpu.async_copy(zb, out_sh.at[dst_l.at[g]], sem, add=True)

    def swait(zb, sem):
        pltpu.make_async_copy(zb, out_sh.at[dst_l.at[0]], sem).wait()

    def dstart(g, pw, sem):
        pltpu.async_copy(pw, den_sh.at[dst_l.at[g]], sem, add=True)

    def dwait(pw, sem):
        pltpu.make_async_copy(pw, den_sh.at[dst_l.at[0]], sem).wait()

    def compute_group(g, zb, pw):
        for c in range(RLEN // 16):
            sl = pl.ds(c * 16, 16)
            gsl = pl.ds(g * RLEN + c * 16, 16)
            srcv = src_l[gsl]
            dstv = dst_l[g, sl]
            prv = pr_l[gsl]
            w1 = plsc.load_gather(s_l, [srcv])
            w2 = plsc.load_gather(s_l, [dstv])
            s1v = plsc.bitcast(lax.shift_left(w1, 16), jnp.float32)
            s2v = plsc.bitcast(w2 & np.int32(-65536), jnp.float32)
            av = s1v + s2v
            ev = jnp.where(av > 0, av, av * np.float32(0.01))
            pv = jnp.exp(prv * ev)
            pw[sl] = pv
            for lane in range(16):
                pb = _lane_bcast(pv, lane)
                j = c * 16 + lane
                for r in range(D // 16):
                    rsl = pl.ds(r * 16, 16)
                    zb[j, rsl] = zb[j, rsl] * pb

    @pl.loop(0, ROWS // SEC)
    def _(sec):
        ebase = wid * EPW + sec * (SEC * RLEN)
        pltpu.sync_copy(dst_hbm.at[pl.ds(ebase, SEC * RLEN)], src_l)

        @pl.loop(0, SEC)
        def _(g):
            for c in range(RLEN // 16):
                dst_l[g, pl.ds(c * 16, 16)] = \
                    src_l[pl.ds(g * RLEN + c * 16, 16)]

        pltpu.sync_copy(src_hbm.at[pl.ds(ebase, SEC * RLEN)], src_l)
        pltpu.sync_copy(pr_hbm.at[pl.ds(ebase, SEC * RLEN)], pr_l)

        gstart(0, zbufa, sga)
        gstart(1, zbufb, sgb)
        gstart(2, zbufc, sgc)

        @pl.loop(0, SEC - 1, step=3)
        def _(g):
            gwait(zbufa, sga)
            compute_group(g, zbufa, prowa)
            sstart(g, zbufa, ssa)
            dstart(g, prowa, sda)

            gwait(zbufb, sgb)
            compute_group(g + 1, zbufb, prowb)
            sstart(g + 1, zbufb, ssb)
            dstart(g + 1, prowb, sdb)

            swait(zbufa, ssa)
            dwait(prowa, sda)
            gstart(g + 3, zbufa, sga)

            gwait(zbufc, sgc)
            compute_group(g + 2, zbufc, prowc)
            sstart(g + 2, zbufc, ssc)
            dstart(g + 2, prowc, sdc)

            @pl.when(g + 4 < SEC)
            def _():
                swait(zbufb, ssb)
                dwait(prowb, sdb)
                gstart(g + 4, zbufb, sgb)

            @pl.when(g + 5 < SEC)
            def _():
                swait(zbufc, ssc)
                dwait(prowc, sdc)
                gstart(g + 5, zbufc, sgc)

        gwait(zbufa, sga)
        compute_group(SEC - 1, zbufa, prowa)
        pltpu.sync_copy(zbufa, out_sh.at[dst_l.at[SEC - 1]], add=True)
        pltpu.sync_copy(prowa, den_sh.at[dst_l.at[SEC - 1]], add=True)
        swait(zbufb, ssb)
        dwait(prowb, sdb)
        swait(zbufc, ssc)
        dwait(prowc, sdc)

    plsc.subcore_barrier()

    @pl.when(sid < 15)
    def _():
        for b in range(8):
            sl = pl.ds(rbase + b * RLEN, RLEN)
            pltpu.sync_copy(out_sh.at[sl], out_hbm.at[cid].at[sl])

    @pl.when(sid == 15)
    def _():
        for b in range(5):
            sl = pl.ds(rbase + b * RLEN, RLEN)
            pltpu.sync_copy(out_sh.at[sl], out_hbm.at[cid].at[sl])

    @pl.when(sid == 0)
    def _():
        pltpu.sync_copy(den_sh, den_hbm.at[cid])


def _sc_edge(z, s, src3, dst3, pr3):
    mesh = plsc.VectorSubcoreMesh(core_axis_name="c", subcore_axis_name="s")
    cp = pltpu.CompilerParams()
    if "needs_layout_passes" in pltpu.CompilerParams.__dataclass_fields__:
        cp = dataclasses.replace(cp, needs_layout_passes=False)
    kern = pl.kernel(
        _sc_body,
        compiler_params=cp,
        out_type=(
            jax.ShapeDtypeStruct((NCORE, N, D), jnp.float32),
            jax.ShapeDtypeStruct((NCORE, N), jnp.float32),
        ),
        mesh=mesh,
        scratch_types=[
            pltpu.VMEM((N,), jnp.int32),
            pltpu.VMEM((SEC * RLEN,), jnp.int32),
            pltpu.VMEM((SEC, RLEN), jnp.int32),
            pltpu.VMEM((SEC * RLEN,), jnp.float32),
            pltpu.VMEM((RLEN,), jnp.float32),
            pltpu.VMEM((RLEN,), jnp.float32),
            pltpu.VMEM((RLEN,), jnp.float32),
            pltpu.VMEM((RLEN, D), jnp.float32),
            pltpu.VMEM((RLEN, D), jnp.float32),
            pltpu.VMEM((RLEN, D), jnp.float32),
            pltpu.SemaphoreType.DMA,
            pltpu.SemaphoreType.DMA,
            pltpu.SemaphoreType.DMA,
            pltpu.SemaphoreType.DMA,
            pltpu.SemaphoreType.DMA,
            pltpu.SemaphoreType.DMA,
            pltpu.SemaphoreType.DMA,
            pltpu.SemaphoreType.DMA,
            pltpu.SemaphoreType.DMA,
            pltpu.VMEM_SHARED((N, D), jnp.float32),
            pltpu.VMEM_SHARED((N,), jnp.float32),
        ],
    )
    return kern(z, s, src3, dst3, pr3)


def _combine_body(p_ref, d_ref, o_ref):
    ps = p_ref[0] + p_ref[1]
    dsum = d_ref[0] + d_ref[1]
    safe = jnp.where(dsum > 0, dsum, np.float32(1.0))
    o_ref[...] = jnp.where(dsum > 0, ps / safe, np.float32(0.0))


def _tc_combine(out_parts, den_parts):
    blk = 1000
    return pl.pallas_call(
        _combine_body,
        grid=(N // blk,),
        in_specs=[
            pl.BlockSpec((NCORE, blk, D), lambda i: (0, i, 0)),
            pl.BlockSpec((NCORE, blk, 1), lambda i: (0, i, 0)),
        ],
        out_specs=pl.BlockSpec((blk, D), lambda i: (i, 0)),
        out_shape=jax.ShapeDtypeStruct((N, D), jnp.float32),
    )(out_parts, den_parts)


def kernel(h, edge_index, dist, fc_W, attn_W):
    src = edge_index[0]
    dst = edge_index[1]
    dist2d = dist.reshape(E // D, D)
    attn_T = attn_W.reshape(2, D).T
    z, s, probs = _tc_prep(h, fc_W, attn_T, dist2d)
    s_pack = lax.bitcast_convert_type(s.astype(jnp.bfloat16), jnp.int32)
    out_parts, den_parts = _sc_edge(z, s_pack, src, dst, probs.reshape(E))
    return _tc_combine(out_parts, den_parts.reshape(NCORE, N, 1))
